# baseline (speedup 1.0000x reference)
import functools

import jax
import jax.numpy as jnp
from jax import lax
from jax.experimental import pallas as pl
from jax.experimental.pallas import tpu as pltpu
from jax.experimental.pallas import tpu_sc as plsc

B = 4096
F = 26
D = 16
DIN = F * D
HOUT = 512
NW = 32
ROWS_PER_W = B * F // NW
CHUNK = 128
NCHUNK = ROWS_PER_W // CHUNK
BM = 512


@functools.cache
def _make_gather():
    mesh = plsc.VectorSubcoreMesh(core_axis_name="c", subcore_axis_name="s")

    @functools.partial(
        pl.kernel,
        mesh=mesh,
        out_type=jax.ShapeDtypeStruct((B * F, D), jnp.float32),
        scratch_types=[
            pltpu.VMEM((NCHUNK, CHUNK), jnp.int32),
            pltpu.VMEM((ROWS_PER_W, D), jnp.float32),
            pltpu.SemaphoreType.DMA,
        ],
        compiler_params=pltpu.CompilerParams(use_tc_tiling_on_sc=False),
    )
    def gather_kernel(idx_hbm, emb_hbm, out_hbm, idx_v, rows_v, sem):
        wid = lax.axis_index("s") * 2 + lax.axis_index("c")
        pltpu.sync_copy(idx_hbm.at[wid], idx_v)

        def fire(j, carry):
            pltpu.async_copy(
                emb_hbm.at[idx_v.at[j]],
                rows_v.at[pl.ds(j * CHUNK, CHUNK)],
                sem,
            )
            return carry

        lax.fori_loop(0, NCHUNK, fire, 0)
        out_slice = out_hbm.at[pl.ds(wid * ROWS_PER_W, ROWS_PER_W)]
        pltpu.make_async_copy(out_slice, rows_v, sem).wait()
        pltpu.sync_copy(rows_v, out_slice)

    return gather_kernel


def _dense_body(x_ref, g_ref, bt_ref, w0, b0, w1, b1, w2, b2, w3, b3, w4, b4,
                cw_ref, cb_ref, fx_ref, fh_ref, fb_ref, out_ref):
    x = x_ref[...]
    mean = jnp.mean(x, axis=1, keepdims=True)
    xc = x - mean
    var = jnp.mean(xc * xc, axis=1, keepdims=True)
    h = xc * lax.rsqrt(var + 1e-5) * g_ref[...] + bt_ref[...]
    for w_r, b_r in ((w0, b0), (w1, b1), (w2, b2), (w3, b3), (w4, b4)):
        h = jnp.maximum(
            jnp.dot(h, w_r[...], preferred_element_type=jnp.float32) + b_r[...],
            0.0,
        )
    xl = x
    for i in range(3):
        xw = jnp.sum(xl * cw_ref[i:i + 1, :], axis=1, keepdims=True)
        xl = x * xw + cb_ref[i:i + 1, :] + xl
    logit = (jnp.sum(xl * fx_ref[...], axis=1, keepdims=True)
             + jnp.sum(h * fh_ref[...], axis=1, keepdims=True)
             + fb_ref[...])
    out_ref[...] = jax.nn.sigmoid(logit)


def _dense_call(x, bn_gamma, bn_beta, W0, b0, W1, b1, W2, b2, W3, b3, W4, b4,
                cross_w, cross_b, fc_w, fc_b):
    grid = (B // BM,)

    def _full(a):
        return pl.BlockSpec(a.shape, lambda i: (0,) * a.ndim)

    weights = (bn_gamma.reshape(1, DIN), bn_beta.reshape(1, DIN),
               W0, b0.reshape(1, -1), W1, b1.reshape(1, -1),
               W2, b2.reshape(1, -1), W3, b3.reshape(1, -1),
               W4, b4.reshape(1, -1),
               cross_w, cross_b,
               fc_w[:DIN, 0].reshape(1, DIN), fc_w[DIN:, 0].reshape(1, HOUT),
               fc_b.reshape(1, 1))
    return pl.pallas_call(
        _dense_body,
        grid=grid,
        in_specs=[pl.BlockSpec((BM, DIN), lambda i: (i, 0))]
        + [_full(w) for w in weights],
        out_specs=pl.BlockSpec((BM, 1), lambda i: (i, 0)),
        out_shape=jax.ShapeDtypeStruct((B, 1), jnp.float32),
        compiler_params=pltpu.CompilerParams(
            dimension_semantics=("arbitrary",),
        ),
    )(x, *weights)


def kernel(indices, emb, bn_gamma, bn_beta, W0, b0, W1, b1, W2, b2, W3, b3,
           W4, b4, cross_w, cross_b, fc_w, fc_b):
    idx = indices.astype(jnp.int32).reshape(NW, NCHUNK, CHUNK)
    gathered = _make_gather()(idx, emb)
    x = gathered.reshape(B, DIN)
    return _dense_call(x, bn_gamma, bn_beta, W0, b0, W1, b1, W2, b2, W3, b3,
                       W4, b4, cross_w, cross_b, fc_w, fc_b)
